# Initial kernel scaffold; baseline (speedup 1.0000x reference)
import jax
import jax.numpy as jnp
from jax.experimental import pallas as pl


def kernel(cfg_nodes_encodings, W_i, W_h, b_i, b_h, permutations, unflattener_mask, lengths):
    raise NotImplementedError("write your pallas kernel here")


# speedup vs baseline: 2.5393x; 2.5393x over previous
import functools

import jax
import jax.numpy as jnp
from jax import lax
from jax.experimental import pallas as pl
from jax.experimental.pallas import tpu as pltpu
from jax.experimental.pallas import tpu_sc as plsc

_NW = 32
_GCH = 64
_SCH = 64
_T = 64


def _mm_body(x_ref, wi_ref, bi_ref, o_ref):
    o_ref[...] = (
        jnp.dot(x_ref[...], wi_ref[...], preferred_element_type=jnp.float32)
        + bi_ref[...]
    )


def _xw_all(x, W_i, b_i):
    total, d = x.shape
    n3 = W_i.shape[1]
    mt = 512
    return pl.pallas_call(
        _mm_body,
        grid=(total // mt,),
        in_specs=[
            pl.BlockSpec((mt, d), lambda i: (i, 0)),
            pl.BlockSpec((d, n3), lambda i: (0, 0)),
            pl.BlockSpec((1, n3), lambda i: (0, 0)),
        ],
        out_specs=pl.BlockSpec((mt, n3), lambda i: (i, 0)),
        out_shape=jax.ShapeDtypeStruct((total, n3), jnp.float32),
    )(x, W_i, b_i.reshape(1, n3))


def _scan_body(xw_ref, wh_ref, bh_ref, len_ref, o_ref, h_ref):
    pid = pl.program_id(0)

    @pl.when(pid == 0)
    def _():
        h_ref[...] = jnp.zeros_like(h_ref)

    wh = wh_ref[...]
    bh = bh_ref[...]
    lens = len_ref[...]
    d = wh.shape[0]

    def step(t, h):
        xw = xw_ref[t]
        hU = jnp.dot(h, wh, preferred_element_type=jnp.float32) + bh
        r = jax.nn.sigmoid(xw[:, :d] + hU[:, :d])
        z = jax.nn.sigmoid(xw[:, d:2 * d] + hU[:, d:2 * d])
        n = jnp.tanh(xw[:, 2 * d:] + r * hU[:, 2 * d:])
        hnew = (1.0 - z) * n + z * h
        m = lens > (pid * _T + t)
        o_ref[t] = jnp.where(m, hnew, 0.0)
        return jnp.where(m, hnew, h)

    h_ref[...] = lax.fori_loop(0, _T, step, h_ref[...])


def _gru_scan(xw_seq, W_h, b_h, lengths):
    l, b, n3 = xw_seq.shape
    d = W_h.shape[0]
    lens = jnp.broadcast_to(lengths.astype(jnp.int32)[:, None], (b, d))
    return pl.pallas_call(
        _scan_body,
        grid=(l // _T,),
        in_specs=[
            pl.BlockSpec((_T, b, n3), lambda i: (i, 0, 0)),
            pl.BlockSpec((d, n3), lambda i: (0, 0)),
            pl.BlockSpec((1, n3), lambda i: (0, 0)),
            pl.BlockSpec((b, d), lambda i: (0, 0)),
        ],
        out_specs=pl.BlockSpec((_T, b, d), lambda i: (i, 0, 0)),
        out_shape=jax.ShapeDtypeStruct((l, b, d), jnp.float32),
        scratch_shapes=[pltpu.VMEM((b, d), jnp.float32)],
        compiler_params=pltpu.CompilerParams(
            dimension_semantics=("arbitrary",)),
    )(xw_seq, W_h, b_h.reshape(1, n3), lens)


def _sc_gather(table, idx):
    nrows = idx.shape[0]
    n3 = table.shape[1]
    rpw = nrows // _NW
    nch = rpw // _GCH
    mesh = plsc.VectorSubcoreMesh(core_axis_name="c", subcore_axis_name="s")

    @functools.partial(
        pl.kernel,
        mesh=mesh,
        out_type=jax.ShapeDtypeStruct((nrows, n3), jnp.float32),
        scratch_types=[
            pltpu.VMEM((rpw,), jnp.int32),
            pltpu.VMEM((_GCH, n3), jnp.float32),
            pltpu.SemaphoreType.DMA,
        ],
    )
    def k(table_hbm, idx_hbm, out_hbm, idx_v, rows_v, sem):
        wid = lax.axis_index("s") * 2 + lax.axis_index("c")
        base = wid * rpw
        pltpu.sync_copy(idx_hbm.at[pl.ds(base, rpw)], idx_v)

        def body(c, carry):
            pltpu.async_copy(
                table_hbm.at[idx_v.at[pl.ds(c * _GCH, _GCH)]], rows_v, sem
            ).wait()
            pltpu.sync_copy(rows_v, out_hbm.at[pl.ds(base + c * _GCH, _GCH)])
            return carry

        lax.fori_loop(0, nch, body, 0)

    return k(table, idx)


def _sc_scatter(rows, idx3, out_rows):
    nrows, d = rows.shape
    rpw = nrows // _NW
    nch = rpw // _SCH
    mesh = plsc.VectorSubcoreMesh(core_axis_name="c", subcore_axis_name="s")

    @functools.partial(
        pl.kernel,
        mesh=mesh,
        out_type=jax.ShapeDtypeStruct((out_rows, d), jnp.float32),
        scratch_types=[
            pltpu.VMEM((nch, _SCH), jnp.int32),
            pltpu.VMEM((_SCH, d), jnp.float32),
            pltpu.SemaphoreType.DMA,
        ],
    )
    def k(rows_hbm, idx_hbm, out_hbm, idx_v, rows_v, sem):
        wid = lax.axis_index("s") * 2 + lax.axis_index("c")
        base = wid * rpw
        pltpu.sync_copy(idx_hbm.at[wid], idx_v)

        def body(c, carry):
            pltpu.sync_copy(rows_hbm.at[pl.ds(base + c * _SCH, _SCH)], rows_v)
            pltpu.async_copy(rows_v, out_hbm.at[idx_v.at[c]], sem).wait()
            return carry

        lax.fori_loop(0, nch, body, 0)

    return k(rows, idx3)


def kernel(cfg_nodes_encodings, W_i, W_h, b_i, b_h, permutations,
           unflattener_mask, lengths):
    x = cfg_nodes_encodings
    total, d = x.shape
    bsz, l = permutations.shape
    n3 = W_i.shape[1]

    perm = permutations.astype(jnp.int32)
    perm_t = perm.T.reshape(-1)
    idx_out = (
        jnp.where(unflattener_mask, perm, total)
        .T.reshape(_NW, (l * bsz) // _NW // _SCH, _SCH)
        .astype(jnp.int32)
    )

    xw_all = _xw_all(x, W_i, b_i)
    xw_seq = _sc_gather(xw_all, perm_t).reshape(l, bsz, n3)
    path = _gru_scan(xw_seq, W_h, b_h, lengths)
    out_pad = _sc_scatter(path.reshape(l * bsz, d), idx_out, total + 8)
    return out_pad[:total]


# speedup vs baseline: 2.5652x; 1.0102x over previous
import functools

import jax
import jax.numpy as jnp
from jax import lax
from jax.experimental import pallas as pl
from jax.experimental.pallas import tpu as pltpu
from jax.experimental.pallas import tpu_sc as plsc

_NW = 32
_GCH = 32
_SCH = 64
_T = 64


def _mm_body(x_ref, wi_ref, bi_ref, o_ref):
    o_ref[...] = (
        jnp.dot(x_ref[...], wi_ref[...], preferred_element_type=jnp.float32)
        + bi_ref[...]
    )


def _xw_all(x, W_i, b_i):
    total, d = x.shape
    n3 = W_i.shape[1]
    mt = 512
    return pl.pallas_call(
        _mm_body,
        grid=(total // mt,),
        in_specs=[
            pl.BlockSpec((mt, d), lambda i: (i, 0)),
            pl.BlockSpec((d, n3), lambda i: (0, 0)),
            pl.BlockSpec((1, n3), lambda i: (0, 0)),
        ],
        out_specs=pl.BlockSpec((mt, n3), lambda i: (i, 0)),
        out_shape=jax.ShapeDtypeStruct((total, n3), jnp.float32),
    )(x, W_i, b_i.reshape(1, n3))


def _scan_body(xw_ref, wh_ref, bh_ref, len_ref, o_ref, h_ref):
    pid = pl.program_id(0)

    @pl.when(pid == 0)
    def _():
        h_ref[...] = jnp.zeros_like(h_ref)

    wh = wh_ref[...]
    bh = bh_ref[...]
    lens = len_ref[...]
    d = wh.shape[0]

    def step(t, h):
        xw = xw_ref[t]
        hU = jnp.dot(h, wh, preferred_element_type=jnp.float32) + bh
        r = jax.nn.sigmoid(xw[:, :d] + hU[:, :d])
        z = jax.nn.sigmoid(xw[:, d:2 * d] + hU[:, d:2 * d])
        n = jnp.tanh(xw[:, 2 * d:] + r * hU[:, 2 * d:])
        hnew = (1.0 - z) * n + z * h
        m = lens > (pid * _T + t)
        o_ref[t] = jnp.where(m, hnew, 0.0)
        return jnp.where(m, hnew, h)

    h_ref[...] = lax.fori_loop(0, _T, step, h_ref[...])


def _gru_scan(xw_seq, W_h, b_h, lengths):
    l, b, n3 = xw_seq.shape
    d = W_h.shape[0]
    lens = jnp.broadcast_to(lengths.astype(jnp.int32)[:, None], (b, d))
    return pl.pallas_call(
        _scan_body,
        grid=(l // _T,),
        in_specs=[
            pl.BlockSpec((_T, b, n3), lambda i: (i, 0, 0)),
            pl.BlockSpec((d, n3), lambda i: (0, 0)),
            pl.BlockSpec((1, n3), lambda i: (0, 0)),
            pl.BlockSpec((b, d), lambda i: (0, 0)),
        ],
        out_specs=pl.BlockSpec((_T, b, d), lambda i: (i, 0, 0)),
        out_shape=jax.ShapeDtypeStruct((l, b, d), jnp.float32),
        scratch_shapes=[pltpu.VMEM((b, d), jnp.float32)],
        compiler_params=pltpu.CompilerParams(
            dimension_semantics=("arbitrary",)),
    )(xw_seq, W_h, b_h.reshape(1, n3), lens)


def _sc_gather(table, idx):
    nrows = idx.shape[0]
    n3 = table.shape[1]
    rpw = nrows // _NW
    nch = rpw // _GCH
    mesh = plsc.VectorSubcoreMesh(core_axis_name="c", subcore_axis_name="s")

    @functools.partial(
        pl.kernel,
        mesh=mesh,
        out_type=jax.ShapeDtypeStruct((nrows, n3), jnp.float32),
        scratch_types=[
            pltpu.VMEM((rpw,), jnp.int32),
            pltpu.VMEM((_GCH, n3), jnp.float32),
            pltpu.VMEM((_GCH, n3), jnp.float32),
            pltpu.SemaphoreType.DMA,
            pltpu.SemaphoreType.DMA,
        ],
    )
    def k(table_hbm, idx_hbm, out_hbm, idx_v, b0, b1, gsem, wsem):
        wid = lax.axis_index("s") * 2 + lax.axis_index("c")
        base = wid * rpw
        pltpu.sync_copy(idx_hbm.at[pl.ds(base, rpw)], idx_v)
        bufs = [b0, b1]
        g = [None] * nch
        w = [None] * nch
        g[0] = pltpu.async_copy(
            table_hbm.at[idx_v.at[pl.ds(0, _GCH)]], bufs[0], gsem)
        for c in range(nch):
            if c + 1 < nch:
                if c >= 1:
                    w[c - 1].wait()
                g[c + 1] = pltpu.async_copy(
                    table_hbm.at[idx_v.at[pl.ds((c + 1) * _GCH, _GCH)]],
                    bufs[(c + 1) % 2], gsem)
            g[c].wait()
            w[c] = pltpu.async_copy(
                bufs[c % 2], out_hbm.at[pl.ds(base + c * _GCH, _GCH)], wsem)
        if nch >= 2:
            w[nch - 2].wait()
        w[nch - 1].wait()

    return k(table, idx)


def _sc_scatter(rows, idx3, out_rows):
    nrows, d = rows.shape
    rpw = nrows // _NW
    nch = rpw // _SCH
    mesh = plsc.VectorSubcoreMesh(core_axis_name="c", subcore_axis_name="s")

    @functools.partial(
        pl.kernel,
        mesh=mesh,
        out_type=jax.ShapeDtypeStruct((out_rows, d), jnp.float32),
        scratch_types=[
            pltpu.VMEM((nch, _SCH), jnp.int32),
            pltpu.VMEM((_SCH, d), jnp.float32),
            pltpu.VMEM((_SCH, d), jnp.float32),
            pltpu.SemaphoreType.DMA,
            pltpu.SemaphoreType.DMA,
        ],
    )
    def k(rows_hbm, idx_hbm, out_hbm, idx_v, b0, b1, rsem, ssem):
        wid = lax.axis_index("s") * 2 + lax.axis_index("c")
        base = wid * rpw
        pltpu.sync_copy(idx_hbm.at[wid], idx_v)
        bufs = [b0, b1]
        r = [None] * nch
        s = [None] * nch
        r[0] = pltpu.async_copy(
            rows_hbm.at[pl.ds(base, _SCH)], bufs[0], rsem)
        for c in range(nch):
            if c + 1 < nch:
                if c >= 1:
                    s[c - 1].wait()
                r[c + 1] = pltpu.async_copy(
                    rows_hbm.at[pl.ds(base + (c + 1) * _SCH, _SCH)],
                    bufs[(c + 1) % 2], rsem)
            r[c].wait()
            s[c] = pltpu.async_copy(bufs[c % 2], out_hbm.at[idx_v.at[c]], ssem)
        if nch >= 2:
            s[nch - 2].wait()
        s[nch - 1].wait()

    return k(rows, idx3)


def kernel(cfg_nodes_encodings, W_i, W_h, b_i, b_h, permutations,
           unflattener_mask, lengths):
    x = cfg_nodes_encodings
    total, d = x.shape
    bsz, l = permutations.shape
    n3 = W_i.shape[1]

    perm = permutations.astype(jnp.int32)
    perm_t = perm.T.reshape(-1)
    idx_out = (
        jnp.where(unflattener_mask, perm, total)
        .T.reshape(_NW, (l * bsz) // _NW // _SCH, _SCH)
        .astype(jnp.int32)
    )

    xw_all = _xw_all(x, W_i, b_i)
    xw_seq = _sc_gather(xw_all, perm_t).reshape(l, bsz, n3)
    path = _gru_scan(xw_seq, W_h, b_h, lengths)
    out_pad = _sc_scatter(path.reshape(l * bsz, d), idx_out, total + 8)
    return out_pad[:total]


# speedup vs baseline: 8.1577x; 3.1802x over previous
import functools

import jax
import jax.numpy as jnp
from jax import lax
from jax.experimental import pallas as pl
from jax.experimental.pallas import tpu as pltpu
from jax.experimental.pallas import tpu_sc as plsc

_NW = 32
_T = 64


def _wid():
    return lax.axis_index("s") * 2 + lax.axis_index("c")


def _build_inv(idx_hbm, idx_v, inv_v, nk):
    pltpu.sync_copy(idx_hbm, idx_v)
    iota = lax.iota(jnp.int32, 16)

    def body(k, carry):
        for c in range(8):
            q = idx_v[k, pl.ds(c * 16, 16)]
            plsc.store_scatter(inv_v, [q], iota + (k * 128 + c * 16))
        return carry

    lax.fori_loop(0, nk, body, 0)


def _sc_scatter_x(x, idx2, nslots):
    total, d = x.shape
    nk = idx2.shape[0]
    rpw = total // _NW
    nq = rpw // 16
    mesh = plsc.VectorSubcoreMesh(core_axis_name="c", subcore_axis_name="s")

    @functools.partial(
        pl.kernel,
        mesh=mesh,
        out_type=jax.ShapeDtypeStruct((nslots, d), jnp.float32),
        scratch_types=[
            pltpu.VMEM((nk, 128), jnp.int32),
            pltpu.VMEM((total + 16,), jnp.int32),
            pltpu.VMEM((rpw, d), jnp.float32),
            pltpu.SemaphoreType.DMA,
            pltpu.SemaphoreType.DMA,
        ],
        compiler_params=pltpu.CompilerParams(needs_layout_passes=False),
    )
    def k(x_hbm, idx_hbm, out_hbm, idx_v, inv_v, buf, rsem, wsem):
        w = _wid()
        rd = pltpu.async_copy(x_hbm.at[pl.ds(w * rpw, rpw)], buf, rsem)
        _build_inv(idx_hbm, idx_v, inv_v, nk)
        rd.wait()
        iota = lax.iota(jnp.int32, 16)
        hs = []
        for c in range(nq):
            q = plsc.load_gather(inv_v, [iota + (w * rpw + c * 16)])
            hs.append(pltpu.async_copy(
                buf.at[pl.ds(c * 16, 16)], out_hbm.at[q], wsem))
        for h in hs:
            h.wait()

    return k(x, idx2)


def _sc_gather_out(path_flat, idx2, total):
    d = path_flat.shape[1]
    nk = idx2.shape[0]
    rpw = total // _NW
    nq = rpw // 16
    mesh = plsc.VectorSubcoreMesh(core_axis_name="c", subcore_axis_name="s")

    @functools.partial(
        pl.kernel,
        mesh=mesh,
        out_type=jax.ShapeDtypeStruct((total, d), jnp.float32),
        scratch_types=[
            pltpu.VMEM((nk, 128), jnp.int32),
            pltpu.VMEM((total + 16,), jnp.int32),
            pltpu.VMEM((rpw, d), jnp.float32),
            pltpu.SemaphoreType.DMA,
        ],
        compiler_params=pltpu.CompilerParams(needs_layout_passes=False),
    )
    def k(path_hbm, idx_hbm, out_hbm, idx_v, inv_v, buf, sem):
        w = _wid()
        _build_inv(idx_hbm, idx_v, inv_v, nk)
        iota = lax.iota(jnp.int32, 16)
        hs = []
        for c in range(nq):
            q = plsc.load_gather(inv_v, [iota + (w * rpw + c * 16)])
            hs.append(pltpu.async_copy(
                path_hbm.at[q], buf.at[pl.ds(c * 16, 16)], sem))
        for h in hs:
            h.wait()
        pltpu.sync_copy(buf, out_hbm.at[pl.ds(w * rpw, rpw)])

    return k(path_flat, idx2)


def _scan_body(u_ref, wi_ref, bi_ref, wh_ref, bh_ref, len_ref, o_ref,
               xw_ref, h_ref):
    pid = pl.program_id(0)

    @pl.when(pid == 0)
    def _():
        h_ref[...] = jnp.zeros_like(h_ref)

    b = o_ref.shape[1]
    d = wh_ref.shape[0]
    u = u_ref[...].reshape(_T * b, d)
    xw_ref[...] = (
        jnp.dot(u, wi_ref[...], preferred_element_type=jnp.float32)
        + bi_ref[...]
    )
    wh = wh_ref[...]
    bh = bh_ref[...]
    lens = len_ref[...]

    def step(t, h):
        xw = xw_ref[pl.ds(t * b, b), :]
        hU = jnp.dot(h, wh, preferred_element_type=jnp.float32) + bh
        r = jax.nn.sigmoid(xw[:, :d] + hU[:, :d])
        z = jax.nn.sigmoid(xw[:, d:2 * d] + hU[:, d:2 * d])
        n = jnp.tanh(xw[:, 2 * d:] + r * hU[:, 2 * d:])
        hnew = (1.0 - z) * n + z * h
        m = lens > (pid * _T + t)
        o_ref[t] = jnp.where(m, hnew, 0.0)
        return jnp.where(m, hnew, h)

    h_ref[...] = lax.fori_loop(0, _T, step, h_ref[...])


def _gru_scan(unflat, W_i, b_i, W_h, b_h, lengths):
    l, b, d = unflat.shape
    n3 = W_i.shape[1]
    lens = jnp.broadcast_to(lengths.astype(jnp.int32)[:, None], (b, d))
    return pl.pallas_call(
        _scan_body,
        grid=(l // _T,),
        in_specs=[
            pl.BlockSpec((_T, b, d), lambda i: (i, 0, 0)),
            pl.BlockSpec((d, n3), lambda i: (0, 0)),
            pl.BlockSpec((1, n3), lambda i: (0, 0)),
            pl.BlockSpec((d, n3), lambda i: (0, 0)),
            pl.BlockSpec((1, n3), lambda i: (0, 0)),
            pl.BlockSpec((b, d), lambda i: (0, 0)),
        ],
        out_specs=pl.BlockSpec((_T, b, d), lambda i: (i, 0, 0)),
        out_shape=jax.ShapeDtypeStruct((l, b, d), jnp.float32),
        scratch_shapes=[
            pltpu.VMEM((_T * b, n3), jnp.float32),
            pltpu.VMEM((b, d), jnp.float32),
        ],
        compiler_params=pltpu.CompilerParams(
            dimension_semantics=("arbitrary",)),
    )(unflat, W_i, b_i.reshape(1, n3), W_h, b_h.reshape(1, n3), lens)


def kernel(cfg_nodes_encodings, W_i, W_h, b_i, b_h, permutations,
           unflattener_mask, lengths):
    x = cfg_nodes_encodings
    total, d = x.shape
    bsz, l = permutations.shape

    perm = permutations.astype(jnp.int32)
    idx2 = jnp.where(unflattener_mask, perm, total).T.reshape(
        (l * bsz) // 128, 128).astype(jnp.int32)

    unflat = _sc_scatter_x(x, idx2, l * bsz)
    path = _gru_scan(unflat.reshape(l, bsz, d), W_i, b_i, W_h, b_h, lengths)
    return _sc_gather_out(path.reshape(l * bsz, d), idx2, total)
